# baseline (speedup 1.0000x reference)
import jax
import jax.numpy as jnp
from jax.experimental import pallas as pl
from jax.experimental.pallas import tpu as pltpu

B = 2
NSC = 200
NST = 150
H = 128
EPS = 1e-5

TI_BI = 20
TI_SC = 20
TI_ST = 25
SB_BI = NSC // TI_BI
SB_SC = NSC // TI_SC
SB_ST = NST // TI_ST
NS_BI = B * SB_BI // 2
NS_SC = B * SB_SC // 2
NS_ST = B * SB_ST // 2
NP1 = max(NS_BI, NS_SC, NS_ST)
P_FIN = NP1
P_2 = P_FIN + 1
T_TOT = P_2 + max(NS_BI, NS_SC, NS_ST)


def _node_lin_kernel(xsc_ref, xst_ref, wsc_ref, bsc_ref, wst_ref, bst_ref,
                     ysc_ref, yst_ref):
    ysc_ref[...] = jnp.dot(xsc_ref[...], wsc_ref[...],
                           preferred_element_type=jnp.float32) + bsc_ref[...]
    yst_ref[...] = jnp.dot(xst_ref[...], wst_ref[...],
                           preferred_element_type=jnp.float32) + bst_ref[...]


def _node_linears(xsc, xst, wsc, bsc, wst, bst):
    nsc, nst = xsc.shape[0], xst.shape[0]
    ksc, kst = wsc.shape[1], wst.shape[1]
    return pl.pallas_call(
        _node_lin_kernel,
        out_shape=[jax.ShapeDtypeStruct((nsc, ksc), jnp.float32),
                   jax.ShapeDtypeStruct((nst, kst), jnp.float32)],
    )(xsc, xst, wsc, bsc, wst, bst)


def _p1_body(s, k, e_ref, ahp, bhc, vrow, cw, bn_ref, bn_row,
             agg_scr, sb, ti, col_scr=None, vcolp=None):
    g = 2 * s + k
    b = g // sb
    ii = g % sb
    s_sum = jnp.zeros((1, H), jnp.float32)
    s_sq = jnp.zeros((1, H), jnp.float32)
    if col_scr is not None:
        col_acc = jnp.zeros(bhc.shape, jnp.float32)
    rows = []
    for tt in range(ti):
        et = e_ref[0, tt]
        en = jnp.dot(et.astype(jnp.bfloat16), cw,
                     preferred_element_type=jnp.float32)
        en = en + bhc + ahp[k * ti + tt:k * ti + tt + 1]
        gate = jax.nn.sigmoid(en)
        s_sum = s_sum + jnp.sum(en, axis=0, keepdims=True)
        s_sq = s_sq + jnp.sum(en * en, axis=0, keepdims=True)
        rows.append(jnp.sum(gate * vrow, axis=0, keepdims=True))
        if col_scr is not None:
            col_acc = col_acc + gate * vcolp[k * ti + tt:k * ti + tt + 1]
    agg_scr[b, ii] = jnp.concatenate(rows, axis=0)
    bn_vals = jnp.concatenate([s_sum, s_sq], axis=0)

    if k == 0:
        @pl.when(s == 0)
        def _():
            bn_ref[bn_row:bn_row + 2] = bn_vals

        @pl.when(s != 0)
        def _():
            bn_ref[bn_row:bn_row + 2] = bn_ref[bn_row:bn_row + 2] + bn_vals
    else:
        bn_ref[bn_row:bn_row + 2] = bn_ref[bn_row:bn_row + 2] + bn_vals

    if col_scr is not None:
        if k == 0:
            @pl.when(ii == 0)
            def _():
                col_scr[b] = col_acc

            @pl.when(ii != 0)
            def _():
                col_scr[b] = col_scr[b] + col_acc
        else:
            col_scr[b] = col_scr[b] + col_acc


def _p2_body(k, e_ref, ahs, bhc, cw, o_ref, scale, ti):
    for tt in range(ti):
        et = e_ref[0, tt]
        en = jnp.dot(et.astype(jnp.bfloat16), cw,
                     preferred_element_type=jnp.float32)
        y = jnp.maximum(en * scale + bhc + ahs[k * ti + tt:k * ti + tt + 1],
                        0.0)
        o_ref[0, k * ti + tt] = et + y


def _bn_affine(bn_ref, bn_row, n_rows, gam, bet):
    inv_n = 1.0 / n_rows
    mean = bn_ref[bn_row:bn_row + 1] * inv_n
    var = bn_ref[bn_row + 1:bn_row + 2] * inv_n - mean * mean
    scale = jax.lax.rsqrt(var + EPS) * gam
    shift = bet - mean * scale
    return scale, shift


def _mega_kernel(bi_e0_ref, bi_e1_ref, sc_e0_ref, sc_e1_ref,
                 st_e0_ref, st_e1_ref,
                 bi_ah_ref, bi_bh_ref, bi_vr_ref, bi_vc_ref,
                 sc_ah_ref, sc_bh_ref, sc_vr_ref,
                 st_ah_ref, st_bh_ref, st_vr_ref,
                 uhsc_ref, uhst_ref, hsc_in_ref, hst_in_ref,
                 cw3_ref, cb3_ref, gb_ref,
                 hsc_o_ref, hst_o_ref, bi_o_ref, sc_o_ref, st_o_ref,
                 agg_bi, agg_sc, agg_st, col_bi, bn_scr):
    t = pl.program_id(0)
    cw_bi = cw3_ref[0].astype(jnp.bfloat16)
    cw_sc = cw3_ref[1].astype(jnp.bfloat16)
    cw_st = cw3_ref[2].astype(jnp.bfloat16)
    ne_g = gb_ref[0:1]
    ne_b = gb_ref[1:2]

    @pl.when(t < NS_BI)
    def _():
        bhc = bi_bh_ref[0] + cb3_ref[0:1]
        ahp = bi_ah_ref[0, 0]
        vcolp = bi_vc_ref[0, 0]
        vrow = bi_vr_ref[0]
        for k, e_ref in ((0, bi_e0_ref), (1, bi_e1_ref)):
            _p1_body(t, k, e_ref, ahp, bhc, vrow, cw_bi, bn_scr, 0,
                     agg_bi, SB_BI, TI_BI, col_scr=col_bi, vcolp=vcolp)

    @pl.when(t < NS_SC)
    def _():
        bhc = sc_bh_ref[0] + cb3_ref[1:2]
        ahp = sc_ah_ref[0, 0]
        vrow = sc_vr_ref[0]
        for k, e_ref in ((0, sc_e0_ref), (1, sc_e1_ref)):
            _p1_body(t, k, e_ref, ahp, bhc, vrow, cw_sc, bn_scr, 2,
                     agg_sc, SB_SC, TI_SC)

    @pl.when(t < NS_ST)
    def _():
        bhc = st_bh_ref[0] + cb3_ref[2:3]
        ahp = st_ah_ref[0, 0]
        vrow = st_vr_ref[0]
        for k, e_ref in ((0, st_e0_ref), (1, st_e1_ref)):
            _p1_body(t, k, e_ref, ahp, bhc, vrow, cw_st, bn_scr, 4,
                     agg_st, SB_ST, TI_ST)

    @pl.when(t == P_FIN)
    def _():
        nh_g = gb_ref[2:3]
        nh_b = gb_ref[3:4]

        def finalize(uh_ref, in_ref, out_ref, aggs_fn, n_nodes):
            s1 = jnp.zeros((1, H), jnp.float32)
            s2 = jnp.zeros((1, H), jnp.float32)
            for bb in range(B):
                x = uh_ref[bb] + aggs_fn(bb)
                out_ref[bb] = x
                s1 = s1 + jnp.sum(x, axis=0, keepdims=True)
                s2 = s2 + jnp.sum(x * x, axis=0, keepdims=True)
            n = float(B * n_nodes)
            m = s1 / n
            v = s2 / n - m * m
            sc = jax.lax.rsqrt(v + EPS) * nh_g
            sh = nh_b - m * sc
            for bb in range(B):
                y = jnp.maximum(out_ref[bb] * sc + sh, 0.0)
                out_ref[bb] = in_ref[bb] + y

        def sc_aggs(bb):
            a1 = jnp.concatenate([agg_bi[bb, k] for k in range(SB_BI)], axis=0)
            a2 = jnp.concatenate([agg_sc[bb, k] for k in range(SB_SC)], axis=0)
            return a1 + a2

        def st_aggs(bb):
            a1 = jnp.concatenate([agg_st[bb, k] for k in range(SB_ST)], axis=0)
            return a1 + col_bi[bb]

        finalize(uhsc_ref, hsc_in_ref, hsc_o_ref, sc_aggs, NSC)
        finalize(uhst_ref, hst_in_ref, hst_o_ref, st_aggs, NST)

    @pl.when((t >= P_2) & (t < P_2 + NS_BI))
    def _():
        scale, shift = _bn_affine(bn_scr, 0, float(B * NSC * NST), ne_g, ne_b)
        bhc = (bi_bh_ref[0] + cb3_ref[0:1]) * scale + shift
        ahs = bi_ah_ref[0, 0] * scale
        for k, e_ref in ((0, bi_e0_ref), (1, bi_e1_ref)):
            _p2_body(k, e_ref, ahs, bhc, cw_bi, bi_o_ref, scale, TI_BI)

    @pl.when((t >= P_2) & (t < P_2 + NS_SC))
    def _():
        scale, shift = _bn_affine(bn_scr, 2, float(B * NSC * NSC), ne_g, ne_b)
        bhc = (sc_bh_ref[0] + cb3_ref[1:2]) * scale + shift
        ahs = sc_ah_ref[0, 0] * scale
        for k, e_ref in ((0, sc_e0_ref), (1, sc_e1_ref)):
            _p2_body(k, e_ref, ahs, bhc, cw_sc, sc_o_ref, scale, TI_SC)

    @pl.when((t >= P_2) & (t < P_2 + NS_ST))
    def _():
        scale, shift = _bn_affine(bn_scr, 4, float(B * NST * NST), ne_g, ne_b)
        bhc = (st_bh_ref[0] + cb3_ref[2:3]) * scale + shift
        ahs = st_ah_ref[0, 0] * scale
        for k, e_ref in ((0, st_e0_ref), (1, st_e1_ref)):
            _p2_body(k, e_ref, ahs, bhc, cw_st, st_o_ref, scale, TI_ST)


def _e_idx(k, nsteps, sb):
    def idx(t):
        s = jnp.where(t < P_2,
                      jnp.clip(t, 0, nsteps - 1),
                      jnp.clip(t - P_2, 0, nsteps - 1))
        g = 2 * s + k
        return (g // sb, g % sb, 0, 0)
    return idx


def _pair_idx(nsteps, sb):
    half = sb // 2
    def idx(t):
        s = jnp.where(t < P_2,
                      jnp.clip(t, 0, nsteps - 1),
                      jnp.clip(t - P_2, 0, nsteps - 1))
        return (s // half, s % half, 0, 0)
    return idx


def _b_idx(nsteps, sb):
    half = sb // 2
    def idx(t):
        s = jnp.where(t < P_2,
                      jnp.clip(t, 0, nsteps - 1),
                      jnp.clip(t - P_2, 0, nsteps - 1))
        return (s // half, 0, 0)
    return idx


def _o_idx(nsteps, sb):
    half = sb // 2
    def idx(t):
        s = jnp.clip(t - P_2, 0, nsteps - 1)
        return (s // half, s % half, 0, 0)
    return idx


def kernel(h_sc, h_st, bi_e, bi_graph, sc_e, sc_graph, st_e, st_graph, params):
    p = params

    sc_names = ['U1', 'V1', 'W1', 'bi_A', 'sc_A', 'sc_B']
    st_names = ['U2', 'V2', 'W2', 'bi_B', 'st_A', 'st_B']
    wsc = jnp.concatenate([p[n + '_w'].T for n in sc_names], axis=1)
    bsc = jnp.concatenate([p[n + '_b'] for n in sc_names]).reshape(1, -1)
    wst = jnp.concatenate([p[n + '_w'].T for n in st_names], axis=1)
    bst = jnp.concatenate([p[n + '_b'] for n in st_names]).reshape(1, -1)
    xsc = h_sc.reshape(B * NSC, H)
    xst = h_st.reshape(B * NST, H)
    ysc, yst = _node_linears(xsc, xst, wsc, bsc, wst, bst)
    Uh_sc, Vh_sc, Wh_sc, bi_Ah, sc_Ah, sc_Bh = [
        ysc[:, k * H:(k + 1) * H].reshape(B, NSC, H) for k in range(6)]
    Uh_st, Vh_st, Wh_st, bi_Bh, st_Ah, st_Bh = [
        yst[:, k * H:(k + 1) * H].reshape(B, NST, H) for k in range(6)]

    pair_bi = lambda a: a.reshape(B, SB_BI // 2, 2 * TI_BI, H)
    pair_sc = lambda a: a.reshape(B, SB_SC // 2, 2 * TI_SC, H)
    pair_st = lambda a: a.reshape(B, SB_ST // 2, 2 * TI_ST, H)

    cw3 = jnp.stack([p['bi_C_w'].T, p['sc_C_w'].T, p['st_C_w'].T])
    cb3 = jnp.stack([p['bi_C_b'], p['sc_C_b'], p['st_C_b']])
    gb = jnp.stack([p['ne_g'], p['ne_b'], p['nh_g'], p['nh_b']])

    whole = lambda shape: pl.BlockSpec(shape, lambda t: (0,) * len(shape))
    in_specs = [
        pl.BlockSpec((1, TI_BI, NST, H), _e_idx(0, NS_BI, SB_BI)),
        pl.BlockSpec((1, TI_BI, NST, H), _e_idx(1, NS_BI, SB_BI)),
        pl.BlockSpec((1, TI_SC, NSC, H), _e_idx(0, NS_SC, SB_SC)),
        pl.BlockSpec((1, TI_SC, NSC, H), _e_idx(1, NS_SC, SB_SC)),
        pl.BlockSpec((1, TI_ST, NST, H), _e_idx(0, NS_ST, SB_ST)),
        pl.BlockSpec((1, TI_ST, NST, H), _e_idx(1, NS_ST, SB_ST)),
        pl.BlockSpec((1, 1, 2 * TI_BI, H), _pair_idx(NS_BI, SB_BI)),
        pl.BlockSpec((1, NST, H), _b_idx(NS_BI, SB_BI)),
        pl.BlockSpec((1, NST, H), _b_idx(NS_BI, SB_BI)),
        pl.BlockSpec((1, 1, 2 * TI_BI, H), _pair_idx(NS_BI, SB_BI)),
        pl.BlockSpec((1, 1, 2 * TI_SC, H), _pair_idx(NS_SC, SB_SC)),
        pl.BlockSpec((1, NSC, H), _b_idx(NS_SC, SB_SC)),
        pl.BlockSpec((1, NSC, H), _b_idx(NS_SC, SB_SC)),
        pl.BlockSpec((1, 1, 2 * TI_ST, H), _pair_idx(NS_ST, SB_ST)),
        pl.BlockSpec((1, NST, H), _b_idx(NS_ST, SB_ST)),
        pl.BlockSpec((1, NST, H), _b_idx(NS_ST, SB_ST)),
        whole((B, NSC, H)),
        whole((B, NST, H)),
        whole((B, NSC, H)),
        whole((B, NST, H)),
        whole((3, H, H)),
        whole((3, H)),
        whole((4, H)),
    ]
    out_shape = [
        jax.ShapeDtypeStruct((B, NSC, H), jnp.float32),
        jax.ShapeDtypeStruct((B, NST, H), jnp.float32),
        jax.ShapeDtypeStruct((B, NSC, NST, H), jnp.float32),
        jax.ShapeDtypeStruct((B, NSC, NSC, H), jnp.float32),
        jax.ShapeDtypeStruct((B, NST, NST, H), jnp.float32),
    ]
    out_specs = [
        whole((B, NSC, H)),
        whole((B, NST, H)),
        pl.BlockSpec((1, 2 * TI_BI, NST, H), _o_idx(NS_BI, SB_BI)),
        pl.BlockSpec((1, 2 * TI_SC, NSC, H), _o_idx(NS_SC, SB_SC)),
        pl.BlockSpec((1, 2 * TI_ST, NST, H), _o_idx(NS_ST, SB_ST)),
    ]
    scratch_shapes = [
        pltpu.VMEM((B, SB_BI, TI_BI, H), jnp.float32),
        pltpu.VMEM((B, SB_SC, TI_SC, H), jnp.float32),
        pltpu.VMEM((B, SB_ST, TI_ST, H), jnp.float32),
        pltpu.VMEM((B, NST, H), jnp.float32),
        pltpu.VMEM((6, H), jnp.float32),
    ]
    (hsc_o, hst_o, bi_o, sc_o, st_o) = pl.pallas_call(
        _mega_kernel, grid=(T_TOT,), in_specs=in_specs,
        out_specs=out_specs, out_shape=out_shape,
        scratch_shapes=scratch_shapes)(
        bi_e, bi_e, sc_e, sc_e, st_e, st_e,
        pair_bi(bi_Ah), bi_Bh, Vh_st, pair_bi(Vh_sc),
        pair_sc(sc_Ah), sc_Bh, Wh_sc,
        pair_st(st_Ah), st_Bh, Wh_st,
        Uh_sc, Uh_st, h_sc, h_st,
        cw3, cb3, gb)
    return (hsc_o, hst_o, bi_o, sc_o, st_o)


# speedup vs baseline: 1.0763x; 1.0763x over previous
import jax
import jax.numpy as jnp
from jax.experimental import pallas as pl
from jax.experimental.pallas import tpu as pltpu

B = 2
NSC = 200
NST = 150
H = 128
EPS = 1e-5

TI_BI = 40
TI_SC = 40
TI_ST = 50
SB_BI = NSC // TI_BI
SB_SC = NSC // TI_SC
SB_ST = NST // TI_ST
NB_BI = B * SB_BI
NB_SC = B * SB_SC
NB_ST = B * SB_ST
NP1 = max(NB_BI, NB_SC, NB_ST)
P_FIN = NP1
P_2 = P_FIN + 1
T_TOT = P_2 + max(NB_BI, NB_SC, NB_ST)


def _p1_body(s, e_ref, ah_scr, bh_scr, vr_scr, cw, cb, bn_ref, bn_row,
             agg_scr, sb, ti, col_scr=None, vc_scr=None):
    b = s // sb
    ii = s % sb
    bhc = bh_scr[b] + cb
    vrow = vr_scr[b]
    ah = ah_scr[b, ii]
    s_sum = jnp.zeros((1, H), jnp.float32)
    s_sq = jnp.zeros((1, H), jnp.float32)
    if col_scr is not None:
        vcol = vc_scr[b, ii]
        col_acc = jnp.zeros(bhc.shape, jnp.float32)
    rows = []
    for tt in range(ti):
        et = e_ref[0, tt]
        en = jnp.dot(et.astype(jnp.bfloat16), cw,
                     preferred_element_type=jnp.float32)
        en = en + bhc + ah[tt:tt + 1]
        gate = jax.nn.sigmoid(en)
        s_sum = s_sum + jnp.sum(en, axis=0, keepdims=True)
        s_sq = s_sq + jnp.sum(en * en, axis=0, keepdims=True)
        rows.append(jnp.sum(gate * vrow, axis=0, keepdims=True))
        if col_scr is not None:
            col_acc = col_acc + gate * vcol[tt:tt + 1]
    agg_scr[b, ii] = jnp.concatenate(rows, axis=0)
    bn_vals = jnp.concatenate([s_sum, s_sq], axis=0)

    @pl.when(s == 0)
    def _():
        bn_ref[bn_row:bn_row + 2] = bn_vals

    @pl.when(s != 0)
    def _():
        bn_ref[bn_row:bn_row + 2] = bn_ref[bn_row:bn_row + 2] + bn_vals

    if col_scr is not None:
        @pl.when(ii == 0)
        def _():
            col_scr[b] = col_acc

        @pl.when(ii != 0)
        def _():
            col_scr[b] = col_scr[b] + col_acc


def _p2_body(s, e_ref, ah_scr, bh_scr, cw, cb, o_ref, bn_ref, bn_row,
             n_rows, gam, bet, sb, ti):
    b = s // sb
    ii = s % sb
    inv_n = 1.0 / n_rows
    mean = bn_ref[bn_row:bn_row + 1] * inv_n
    var = bn_ref[bn_row + 1:bn_row + 2] * inv_n - mean * mean
    scale = jax.lax.rsqrt(var + EPS) * gam
    shift = bet - mean * scale
    bhc = (bh_scr[b] + cb) * scale + shift
    ahs = ah_scr[b, ii] * scale
    for tt in range(ti):
        et = e_ref[0, tt]
        en = jnp.dot(et.astype(jnp.bfloat16), cw,
                     preferred_element_type=jnp.float32)
        y = jnp.maximum(en * scale + bhc + ahs[tt:tt + 1], 0.0)
        o_ref[0, tt] = et + y


def _mega_kernel(bi_e_ref, sc_e_ref, st_e_ref,
                 hsc_in_ref, hst_in_ref,
                 wsc_ref, bsc_ref, wst_ref, bst_ref,
                 cw3_ref, cb3_ref, gb_ref,
                 hsc_o_ref, hst_o_ref, bi_o_ref, sc_o_ref, st_o_ref,
                 ah_bi, vc_bi, ah_sc, ah_st,
                 uh_sc, vr_sc, bh_sc, uh_st, vr_bi, bh_bi, vr_st, bh_st,
                 agg_bi, agg_sc, agg_st, col_bi, bn_scr):
    t = pl.program_id(0)
    cw_bi = cw3_ref[0].astype(jnp.bfloat16)
    cw_sc = cw3_ref[1].astype(jnp.bfloat16)
    cw_st = cw3_ref[2].astype(jnp.bfloat16)
    cb_bi = cb3_ref[0:1]
    cb_sc = cb3_ref[1:2]
    cb_st = cb3_ref[2:3]
    ne_g = gb_ref[0:1]
    ne_b = gb_ref[1:2]

    @pl.when(t == 0)
    def _():
        for bb in range(B):
            ysc = jnp.dot(hsc_in_ref[bb], wsc_ref[...],
                          preferred_element_type=jnp.float32) + bsc_ref[...]
            uh_sc[bb] = ysc[:, 0:H]
            vr_sc[bb] = ysc[:, 2 * H:3 * H]
            bh_sc[bb] = ysc[:, 5 * H:6 * H]
            for ii in range(SB_BI):
                r0 = ii * TI_BI
                vc_bi[bb, ii] = ysc[r0:r0 + TI_BI, H:2 * H]
                ah_bi[bb, ii] = ysc[r0:r0 + TI_BI, 3 * H:4 * H]
                ah_sc[bb, ii] = ysc[r0:r0 + TI_SC, 4 * H:5 * H]
            yst = jnp.dot(hst_in_ref[bb], wst_ref[...],
                          preferred_element_type=jnp.float32) + bst_ref[...]
            uh_st[bb] = yst[:, 0:H]
            vr_bi[bb] = yst[:, H:2 * H]
            vr_st[bb] = yst[:, 2 * H:3 * H]
            bh_bi[bb] = yst[:, 3 * H:4 * H]
            bh_st[bb] = yst[:, 5 * H:6 * H]
            for ii in range(SB_ST):
                r0 = ii * TI_ST
                ah_st[bb, ii] = yst[r0:r0 + TI_ST, 4 * H:5 * H]

    @pl.when(t < NB_BI)
    def _():
        _p1_body(t, bi_e_ref, ah_bi, bh_bi, vr_bi, cw_bi, cb_bi, bn_scr, 0,
                 agg_bi, SB_BI, TI_BI, col_scr=col_bi, vc_scr=vc_bi)

    @pl.when(t < NB_SC)
    def _():
        _p1_body(t, sc_e_ref, ah_sc, bh_sc, vr_sc, cw_sc, cb_sc, bn_scr, 2,
                 agg_sc, SB_SC, TI_SC)

    @pl.when(t < NB_ST)
    def _():
        _p1_body(t, st_e_ref, ah_st, bh_st, vr_st, cw_st, cb_st, bn_scr, 4,
                 agg_st, SB_ST, TI_ST)

    @pl.when(t == P_FIN)
    def _():
        nh_g = gb_ref[2:3]
        nh_b = gb_ref[3:4]

        def finalize(uh_scr, in_ref, out_ref, aggs_fn, n_nodes):
            s1 = jnp.zeros((1, H), jnp.float32)
            s2 = jnp.zeros((1, H), jnp.float32)
            for bb in range(B):
                x = uh_scr[bb] + aggs_fn(bb)
                out_ref[bb] = x
                s1 = s1 + jnp.sum(x, axis=0, keepdims=True)
                s2 = s2 + jnp.sum(x * x, axis=0, keepdims=True)
            n = float(B * n_nodes)
            m = s1 / n
            v = s2 / n - m * m
            sc = jax.lax.rsqrt(v + EPS) * nh_g
            sh = nh_b - m * sc
            for bb in range(B):
                y = jnp.maximum(out_ref[bb] * sc + sh, 0.0)
                out_ref[bb] = in_ref[bb] + y

        def sc_aggs(bb):
            a1 = jnp.concatenate([agg_bi[bb, k] for k in range(SB_BI)], axis=0)
            a2 = jnp.concatenate([agg_sc[bb, k] for k in range(SB_SC)], axis=0)
            return a1 + a2

        def st_aggs(bb):
            a1 = jnp.concatenate([agg_st[bb, k] for k in range(SB_ST)], axis=0)
            return a1 + col_bi[bb]

        finalize(uh_sc, hsc_in_ref, hsc_o_ref, sc_aggs, NSC)
        finalize(uh_st, hst_in_ref, hst_o_ref, st_aggs, NST)

    @pl.when((t >= P_2) & (t < P_2 + NB_BI))
    def _():
        _p2_body(t - P_2, bi_e_ref, ah_bi, bh_bi, cw_bi, cb_bi, bi_o_ref,
                 bn_scr, 0, float(B * NSC * NST), ne_g, ne_b, SB_BI, TI_BI)

    @pl.when((t >= P_2) & (t < P_2 + NB_SC))
    def _():
        _p2_body(t - P_2, sc_e_ref, ah_sc, bh_sc, cw_sc, cb_sc, sc_o_ref,
                 bn_scr, 2, float(B * NSC * NSC), ne_g, ne_b, SB_SC, TI_SC)

    @pl.when((t >= P_2) & (t < P_2 + NB_ST))
    def _():
        _p2_body(t - P_2, st_e_ref, ah_st, bh_st, cw_st, cb_st, st_o_ref,
                 bn_scr, 4, float(B * NST * NST), ne_g, ne_b, SB_ST, TI_ST)


def _walk2(nsteps, sb):
    def idx(t):
        s = jnp.where(t < P_2,
                      jnp.clip(t, 0, nsteps - 1),
                      jnp.clip(t - P_2, 0, nsteps - 1))
        return (s // sb, s % sb, 0, 0)
    return idx


def _walk1(nsteps, sb):
    def idx(t):
        s = jnp.clip(t - P_2, 0, nsteps - 1)
        return (s // sb, s % sb, 0, 0)
    return idx


def kernel(h_sc, h_st, bi_e, bi_graph, sc_e, sc_graph, st_e, st_graph, params):
    p = params

    sc_names = ['U1', 'V1', 'W1', 'bi_A', 'sc_A', 'sc_B']
    st_names = ['U2', 'V2', 'W2', 'bi_B', 'st_A', 'st_B']
    wsc = jnp.concatenate([p[n + '_w'].T for n in sc_names], axis=1)
    bsc = jnp.concatenate([p[n + '_b'] for n in sc_names]).reshape(1, -1)
    wst = jnp.concatenate([p[n + '_w'].T for n in st_names], axis=1)
    bst = jnp.concatenate([p[n + '_b'] for n in st_names]).reshape(1, -1)
    cw3 = jnp.stack([p['bi_C_w'].T, p['sc_C_w'].T, p['st_C_w'].T])
    cb3 = jnp.stack([p['bi_C_b'], p['sc_C_b'], p['st_C_b']])
    gb = jnp.stack([p['ne_g'], p['ne_b'], p['nh_g'], p['nh_b']])

    whole = lambda shape: pl.BlockSpec(shape, lambda t: (0,) * len(shape))
    in_specs = [
        pl.BlockSpec((1, TI_BI, NST, H), _walk2(NB_BI, SB_BI)),
        pl.BlockSpec((1, TI_SC, NSC, H), _walk2(NB_SC, SB_SC)),
        pl.BlockSpec((1, TI_ST, NST, H), _walk2(NB_ST, SB_ST)),
        whole((B, NSC, H)),
        whole((B, NST, H)),
        whole((H, 6 * H)),
        whole((1, 6 * H)),
        whole((H, 6 * H)),
        whole((1, 6 * H)),
        whole((3, H, H)),
        whole((3, H)),
        whole((4, H)),
    ]
    out_shape = [
        jax.ShapeDtypeStruct((B, NSC, H), jnp.float32),
        jax.ShapeDtypeStruct((B, NST, H), jnp.float32),
        jax.ShapeDtypeStruct((B, NSC, NST, H), jnp.float32),
        jax.ShapeDtypeStruct((B, NSC, NSC, H), jnp.float32),
        jax.ShapeDtypeStruct((B, NST, NST, H), jnp.float32),
    ]
    out_specs = [
        whole((B, NSC, H)),
        whole((B, NST, H)),
        pl.BlockSpec((1, TI_BI, NST, H), _walk1(NB_BI, SB_BI)),
        pl.BlockSpec((1, TI_SC, NSC, H), _walk1(NB_SC, SB_SC)),
        pl.BlockSpec((1, TI_ST, NST, H), _walk1(NB_ST, SB_ST)),
    ]
    scratch_shapes = [
        pltpu.VMEM((B, SB_BI, TI_BI, H), jnp.float32),
        pltpu.VMEM((B, SB_BI, TI_BI, H), jnp.float32),
        pltpu.VMEM((B, SB_SC, TI_SC, H), jnp.float32),
        pltpu.VMEM((B, SB_ST, TI_ST, H), jnp.float32),
        pltpu.VMEM((B, NSC, H), jnp.float32),
        pltpu.VMEM((B, NSC, H), jnp.float32),
        pltpu.VMEM((B, NSC, H), jnp.float32),
        pltpu.VMEM((B, NST, H), jnp.float32),
        pltpu.VMEM((B, NST, H), jnp.float32),
        pltpu.VMEM((B, NST, H), jnp.float32),
        pltpu.VMEM((B, NST, H), jnp.float32),
        pltpu.VMEM((B, NST, H), jnp.float32),
        pltpu.VMEM((B, SB_BI, TI_BI, H), jnp.float32),
        pltpu.VMEM((B, SB_SC, TI_SC, H), jnp.float32),
        pltpu.VMEM((B, SB_ST, TI_ST, H), jnp.float32),
        pltpu.VMEM((B, NST, H), jnp.float32),
        pltpu.VMEM((6, H), jnp.float32),
    ]
    (hsc_o, hst_o, bi_o, sc_o, st_o) = pl.pallas_call(
        _mega_kernel, grid=(T_TOT,), in_specs=in_specs,
        out_specs=out_specs, out_shape=out_shape,
        scratch_shapes=scratch_shapes)(
        bi_e, sc_e, st_e, h_sc, h_st,
        wsc, bsc, wst, bst, cw3, cb3, gb)
    return (hsc_o, hst_o, bi_o, sc_o, st_o)
